# Initial kernel scaffold; baseline (speedup 1.0000x reference)
import jax
import jax.numpy as jnp
from jax.experimental import pallas as pl


def kernel(x, edge_index, edge_attr, W1, b1, W2, b2, Wu, bu):
    raise NotImplementedError("write your pallas kernel here")


# speedup vs baseline: 6.6193x; 6.6193x over previous
import functools

import jax
import jax.numpy as jnp
from jax import lax
from jax.experimental import pallas as pl
from jax.experimental.pallas import tpu as pltpu
from jax.experimental.pallas import tpu_sc as plsc

N = 10000
E = 320000
D = 128
DE = 16
H = 128
M = 64
MW = 128
DOUT = 128

NC = 2
NS = 16
NW = NC * NS
GSZ = 128
G = E // GSZ
NPAD = 10240
NPT = NPAD // NS


_SC_MESH = plsc.VectorSubcoreMesh(
    core_axis_name="c", subcore_axis_name="s", num_cores=NC, num_subcores=NS
)


def _gather_body(x_hbm, ei0_hbm, ei1_hbm, src_hbm, tgt_hbm,
                 idx0, idx1, buf0, buf1, sem0, sem1):
    wid = lax.axis_index("c") * NS + lax.axis_index("s")
    ngroups = (G + NW - 1) // NW

    @pl.loop(0, ngroups)
    def _(j):
        g = wid + NW * j

        @pl.when(g < G)
        def _():
            e0 = g * GSZ
            pltpu.sync_copy(ei0_hbm.at[pl.ds(e0, GSZ)], idx0)
            pltpu.sync_copy(ei1_hbm.at[pl.ds(e0, GSZ)], idx1)
            ca = pltpu.async_copy(x_hbm.at[idx0], buf0, sem0)
            cb = pltpu.async_copy(x_hbm.at[idx1], buf1, sem1)
            ca.wait()
            cb.wait()
            wa = pltpu.async_copy(buf0, src_hbm.at[pl.ds(e0, GSZ)], sem0)
            wb = pltpu.async_copy(buf1, tgt_hbm.at[pl.ds(e0, GSZ)], sem1)
            wa.wait()
            wb.wait()


_sc_gather = functools.partial(
    pl.kernel,
    out_type=(
        jax.ShapeDtypeStruct((E, D), jnp.float32),
        jax.ShapeDtypeStruct((E, D), jnp.float32),
    ),
    mesh=_SC_MESH,
    scratch_types=[
        pltpu.VMEM((GSZ,), jnp.int32),
        pltpu.VMEM((GSZ,), jnp.int32),
        pltpu.VMEM((GSZ, D), jnp.float32),
        pltpu.VMEM((GSZ, D), jnp.float32),
        pltpu.SemaphoreType.DMA,
        pltpu.SemaphoreType.DMA,
    ],
)(_gather_body)


def _mlp_body(src_ref, tgt_ref, ea_ref, w1a_ref, w1b_ref, w1c_ref, b1_ref,
              w2_ref, b2_ref, mf_ref, mr_ref):
    diff = tgt_ref[...] - src_ref[...]
    d2 = diff * diff
    t1 = jnp.dot(diff, w1a_ref[...], preferred_element_type=jnp.float32)
    t2 = jnp.dot(d2, w1b_ref[...], preferred_element_type=jnp.float32)
    t3 = jnp.dot(ea_ref[...], w1c_ref[...], preferred_element_type=jnp.float32)
    even = t2 + b1_ref[...]
    odd = t1 + t3
    hf = jnp.maximum(even + odd, 0.0)
    hr = jnp.maximum(even - odd, 0.0)
    b2 = b2_ref[...]
    w2 = w2_ref[...]
    mf = jnp.maximum(jnp.dot(hf, w2, preferred_element_type=jnp.float32) + b2, 0.0)
    mr = jnp.maximum(jnp.dot(hr, w2, preferred_element_type=jnp.float32) + b2, 0.0)
    be = mf.shape[0]
    lane = jax.lax.broadcasted_iota(jnp.int32, (be, MW - M), 1)
    pad = jnp.where(lane == 0, 1.0, 0.0).astype(jnp.float32)
    mf_ref[...] = jnp.concatenate([mf, pad], axis=1)
    mr_ref[...] = jnp.concatenate([mr, pad], axis=1)


def _tc_mlp(src, tgt, ea, w1a, w1b, w1c, b1, w2, b2):
    BE = 1280
    nblk = E // BE
    return pl.pallas_call(
        _mlp_body,
        grid=(nblk,),
        in_specs=[
            pl.BlockSpec((BE, D), lambda i: (i, 0)),
            pl.BlockSpec((BE, D), lambda i: (i, 0)),
            pl.BlockSpec((BE, DE), lambda i: (i, 0)),
            pl.BlockSpec((D, H), lambda i: (0, 0)),
            pl.BlockSpec((D, H), lambda i: (0, 0)),
            pl.BlockSpec((DE, H), lambda i: (0, 0)),
            pl.BlockSpec((1, H), lambda i: (0, 0)),
            pl.BlockSpec((H, M), lambda i: (0, 0)),
            pl.BlockSpec((1, M), lambda i: (0, 0)),
        ],
        out_specs=[
            pl.BlockSpec((BE, MW), lambda i: (i, 0)),
            pl.BlockSpec((BE, MW), lambda i: (i, 0)),
        ],
        out_shape=[
            jax.ShapeDtypeStruct((E, MW), jnp.float32),
            jax.ShapeDtypeStruct((E, MW), jnp.float32),
        ],
    )(src, tgt, ea, w1a, w1b, w1c, b1, w2, b2)


def _scatter_body(mf_hbm, mr_hbm, ei0_hbm, ei1_hbm, z_hbm,
                  s_hbm, s_sh, idx0, idx1, vf, vr):
    c = lax.axis_index("c")
    s = lax.axis_index("s")

    pltpu.sync_copy(z_hbm.at[pl.ds(s * NPT, NPT)], s_sh.at[pl.ds(s * NPT, NPT)])
    plsc.subcore_barrier()

    gpc = G // NC
    ngroups = (gpc + NS - 1) // NS

    @pl.loop(0, ngroups)
    def _(j):
        gl = s + NS * j

        @pl.when(gl < gpc)
        def _():
            g = c * gpc + gl
            e0 = g * GSZ
            pltpu.sync_copy(ei0_hbm.at[pl.ds(e0, GSZ)], idx0)
            pltpu.sync_copy(ei1_hbm.at[pl.ds(e0, GSZ)], idx1)
            pltpu.sync_copy(mf_hbm.at[pl.ds(e0, GSZ)], vf)
            pltpu.sync_copy(mr_hbm.at[pl.ds(e0, GSZ)], vr)
            pltpu.sync_copy(vf, s_sh.at[idx1], add=True)
            pltpu.sync_copy(vr, s_sh.at[idx0], add=True)

    plsc.subcore_barrier()
    pltpu.sync_copy(s_sh.at[pl.ds(s * NPT, NPT)], s_hbm.at[c, pl.ds(s * NPT, NPT)])


_sc_scatter = functools.partial(
    pl.kernel,
    out_type=jax.ShapeDtypeStruct((NC, NPAD, MW), jnp.float32),
    mesh=_SC_MESH,
    scratch_types=[
        pltpu.VMEM_SHARED((NPAD, MW), jnp.float32),
        pltpu.VMEM((GSZ,), jnp.int32),
        pltpu.VMEM((GSZ,), jnp.int32),
        pltpu.VMEM((GSZ, MW), jnp.float32),
        pltpu.VMEM((GSZ, MW), jnp.float32),
    ],
)(_scatter_body)


def _final_body(x_ref, sp_ref, wua_ref, wub_ref, wuc_ref, bu_ref, out_ref):
    x = x_ref[...]
    acc = sp_ref[0] + sp_ref[1]
    ssum = acc[:, :M]
    cnt = jnp.maximum(acc[:, M:M + 1], 1.0)
    aggr = ssum / cnt
    out = jnp.dot(x, wua_ref[...], preferred_element_type=jnp.float32)
    out += jnp.dot(x * x, wub_ref[...], preferred_element_type=jnp.float32)
    out += jnp.dot(aggr, wuc_ref[...], preferred_element_type=jnp.float32)
    out_ref[...] = out + bu_ref[...]


def _tc_final(x, sp, wua, wub, wuc, bu):
    return pl.pallas_call(
        _final_body,
        out_shape=jax.ShapeDtypeStruct((N, DOUT), jnp.float32),
    )(x, sp, wua, wub, wuc, bu)


def kernel(x, edge_index, edge_attr, W1, b1, W2, b2, Wu, bu):
    ei0 = edge_index[0]
    ei1 = edge_index[1]

    w1a = W1[:D]
    w1b = W1[D:2 * D]
    w1c = W1[2 * D:]
    b1r = b1.reshape(1, H)
    b2r = b2.reshape(1, M)

    src, tgt = _sc_gather(x, ei0, ei1)
    mf, mr = _tc_mlp(src, tgt, edge_attr, w1a, w1b, w1c, b1r, W2, b2r)

    z = jnp.zeros((NPAD, MW), jnp.float32)
    sp = _sc_scatter(mf, mr, ei0, ei1, z)
    sp = sp[:, :N]

    wua = Wu[:D]
    wub = Wu[D:2 * D]
    wuc = Wu[2 * D:]
    bur = bu.reshape(1, DOUT)
    return _tc_final(x, sp, wua, wub, wuc, bur)


# speedup vs baseline: 7.0350x; 1.0628x over previous
import functools

import jax
import jax.numpy as jnp
from jax import lax
from jax.experimental import pallas as pl
from jax.experimental.pallas import tpu as pltpu
from jax.experimental.pallas import tpu_sc as plsc

N = 10000
E = 320000
D = 128
DE = 16
H = 128
M = 64
MW = 128
DOUT = 128

NC = 2
NS = 16
NW = NC * NS
GSZ = 128
G = E // GSZ
NPAD = 10240
NPT = NPAD // NS


_SC_MESH = plsc.VectorSubcoreMesh(
    core_axis_name="c", subcore_axis_name="s", num_cores=NC, num_subcores=NS
)


XCH = 632


def _gather_body(x_hbm, ei0_hbm, ei1_hbm, src_hbm, tgt_hbm,
                 x_sh, idx0, idx1, buf0, buf1, sem0, sem1):
    s = lax.axis_index("s")
    wid = lax.axis_index("c") * NS + s

    @pl.when(s < NS - 1)
    def _():
        pltpu.sync_copy(x_hbm.at[pl.ds(s * XCH, XCH)],
                        x_sh.at[pl.ds(s * XCH, XCH)])

    @pl.when(s == NS - 1)
    def _():
        pltpu.sync_copy(x_hbm.at[pl.ds((NS - 1) * XCH, N - (NS - 1) * XCH)],
                        x_sh.at[pl.ds((NS - 1) * XCH, N - (NS - 1) * XCH)])

    plsc.subcore_barrier()

    ngroups = (G + NW - 1) // NW

    @pl.loop(0, ngroups)
    def _(j):
        g = wid + NW * j

        @pl.when(g < G)
        def _():
            e0 = g * GSZ
            pltpu.sync_copy(ei0_hbm.at[pl.ds(e0, GSZ)], idx0)
            pltpu.sync_copy(ei1_hbm.at[pl.ds(e0, GSZ)], idx1)
            ca = pltpu.async_copy(x_sh.at[idx0], buf0, sem0)
            cb = pltpu.async_copy(x_sh.at[idx1], buf1, sem1)
            ca.wait()
            cb.wait()
            wa = pltpu.async_copy(buf0, src_hbm.at[pl.ds(e0, GSZ)], sem0)
            wb = pltpu.async_copy(buf1, tgt_hbm.at[pl.ds(e0, GSZ)], sem1)
            wa.wait()
            wb.wait()


_sc_gather = functools.partial(
    pl.kernel,
    out_type=(
        jax.ShapeDtypeStruct((E, D), jnp.float32),
        jax.ShapeDtypeStruct((E, D), jnp.float32),
    ),
    mesh=_SC_MESH,
    scratch_types=[
        pltpu.VMEM_SHARED((N, D), jnp.float32),
        pltpu.VMEM((GSZ,), jnp.int32),
        pltpu.VMEM((GSZ,), jnp.int32),
        pltpu.VMEM((GSZ, D), jnp.float32),
        pltpu.VMEM((GSZ, D), jnp.float32),
        pltpu.SemaphoreType.DMA,
        pltpu.SemaphoreType.DMA,
    ],
)(_gather_body)


def _mlp_body(src_ref, tgt_ref, ea_ref, w1a_ref, w1b_ref, w1c_ref, b1_ref,
              w2_ref, b2_ref, mf_ref, mr_ref):
    diff = tgt_ref[...] - src_ref[...]
    d2 = diff * diff
    t1 = jnp.dot(diff, w1a_ref[...], preferred_element_type=jnp.float32)
    t2 = jnp.dot(d2, w1b_ref[...], preferred_element_type=jnp.float32)
    t3 = jnp.dot(ea_ref[...], w1c_ref[...], preferred_element_type=jnp.float32)
    even = t2 + b1_ref[...]
    odd = t1 + t3
    hf = jnp.maximum(even + odd, 0.0)
    hr = jnp.maximum(even - odd, 0.0)
    b2 = b2_ref[...]
    w2 = w2_ref[...]
    mf = jnp.maximum(jnp.dot(hf, w2, preferred_element_type=jnp.float32) + b2, 0.0)
    mr = jnp.maximum(jnp.dot(hr, w2, preferred_element_type=jnp.float32) + b2, 0.0)
    be = mf.shape[0]
    lane = jax.lax.broadcasted_iota(jnp.int32, (be, MW - M), 1)
    pad = jnp.where(lane == 0, 1.0, 0.0).astype(jnp.float32)
    mf_ref[...] = jnp.concatenate([mf, pad], axis=1)
    mr_ref[...] = jnp.concatenate([mr, pad], axis=1)


def _tc_mlp(src, tgt, ea, w1a, w1b, w1c, b1, w2, b2):
    BE = 1280
    nblk = E // BE
    return pl.pallas_call(
        _mlp_body,
        grid=(nblk,),
        in_specs=[
            pl.BlockSpec((BE, D), lambda i: (i, 0)),
            pl.BlockSpec((BE, D), lambda i: (i, 0)),
            pl.BlockSpec((BE, DE), lambda i: (i, 0)),
            pl.BlockSpec((D, H), lambda i: (0, 0)),
            pl.BlockSpec((D, H), lambda i: (0, 0)),
            pl.BlockSpec((DE, H), lambda i: (0, 0)),
            pl.BlockSpec((1, H), lambda i: (0, 0)),
            pl.BlockSpec((H, M), lambda i: (0, 0)),
            pl.BlockSpec((1, M), lambda i: (0, 0)),
        ],
        out_specs=[
            pl.BlockSpec((BE, MW), lambda i: (i, 0)),
            pl.BlockSpec((BE, MW), lambda i: (i, 0)),
        ],
        out_shape=[
            jax.ShapeDtypeStruct((E, MW), jnp.float32),
            jax.ShapeDtypeStruct((E, MW), jnp.float32),
        ],
    )(src, tgt, ea, w1a, w1b, w1c, b1, w2, b2)


def _scatter_body(mf_hbm, mr_hbm, ei0_hbm, ei1_hbm, z_hbm,
                  s_hbm, s_sh, idx0, idx1, vf, vr):
    c = lax.axis_index("c")
    s = lax.axis_index("s")

    pltpu.sync_copy(z_hbm.at[pl.ds(s * NPT, NPT)], s_sh.at[pl.ds(s * NPT, NPT)])
    plsc.subcore_barrier()

    gpc = G // NC
    ngroups = (gpc + NS - 1) // NS

    @pl.loop(0, ngroups)
    def _(j):
        gl = s + NS * j

        @pl.when(gl < gpc)
        def _():
            g = c * gpc + gl
            e0 = g * GSZ
            pltpu.sync_copy(ei0_hbm.at[pl.ds(e0, GSZ)], idx0)
            pltpu.sync_copy(ei1_hbm.at[pl.ds(e0, GSZ)], idx1)
            pltpu.sync_copy(mf_hbm.at[pl.ds(e0, GSZ)], vf)
            pltpu.sync_copy(mr_hbm.at[pl.ds(e0, GSZ)], vr)
            pltpu.sync_copy(vf, s_sh.at[idx1], add=True)
            pltpu.sync_copy(vr, s_sh.at[idx0], add=True)

    plsc.subcore_barrier()
    pltpu.sync_copy(s_sh.at[pl.ds(s * NPT, NPT)], s_hbm.at[c, pl.ds(s * NPT, NPT)])


_sc_scatter = functools.partial(
    pl.kernel,
    out_type=jax.ShapeDtypeStruct((NC, NPAD, MW), jnp.float32),
    mesh=_SC_MESH,
    scratch_types=[
        pltpu.VMEM_SHARED((NPAD, MW), jnp.float32),
        pltpu.VMEM((GSZ,), jnp.int32),
        pltpu.VMEM((GSZ,), jnp.int32),
        pltpu.VMEM((GSZ, MW), jnp.float32),
        pltpu.VMEM((GSZ, MW), jnp.float32),
    ],
)(_scatter_body)


def _final_body(x_ref, sp_ref, wua_ref, wub_ref, wuc_ref, bu_ref, out_ref):
    x = x_ref[...]
    acc = sp_ref[0] + sp_ref[1]
    ssum = acc[:, :M]
    cnt = jnp.maximum(acc[:, M:M + 1], 1.0)
    aggr = ssum / cnt
    out = jnp.dot(x, wua_ref[...], preferred_element_type=jnp.float32)
    out += jnp.dot(x * x, wub_ref[...], preferred_element_type=jnp.float32)
    out += jnp.dot(aggr, wuc_ref[...], preferred_element_type=jnp.float32)
    out_ref[...] = out + bu_ref[...]


def _tc_final(x, sp, wua, wub, wuc, bu):
    return pl.pallas_call(
        _final_body,
        out_shape=jax.ShapeDtypeStruct((N, DOUT), jnp.float32),
    )(x, sp, wua, wub, wuc, bu)


def kernel(x, edge_index, edge_attr, W1, b1, W2, b2, Wu, bu):
    ei0 = edge_index[0]
    ei1 = edge_index[1]

    w1a = W1[:D]
    w1b = W1[D:2 * D]
    w1c = W1[2 * D:]
    b1r = b1.reshape(1, H)
    b2r = b2.reshape(1, M)

    src, tgt = _sc_gather(x, ei0, ei1)
    mf, mr = _tc_mlp(src, tgt, edge_attr, w1a, w1b, w1c, b1r, W2, b2r)

    z = jnp.zeros((NPAD, MW), jnp.float32)
    sp = _sc_scatter(mf, mr, ei0, ei1, z)
    sp = sp[:, :N]

    wua = Wu[:D]
    wub = Wu[D:2 * D]
    wuc = Wu[2 * D:]
    bur = bu.reshape(1, DOUT)
    return _tc_final(x, sp, wua, wub, wuc, bur)


# speedup vs baseline: 7.6460x; 1.0869x over previous
import functools

import jax
import jax.numpy as jnp
from jax import lax
from jax.experimental import pallas as pl
from jax.experimental.pallas import tpu as pltpu
from jax.experimental.pallas import tpu_sc as plsc

N = 10000
E = 320000
D = 128
DE = 16
H = 128
M = 64
MW = 128
DOUT = 128

NC = 2
NS = 16
NW = NC * NS
GSZ = 128
G = E // GSZ
NPAD = 10240
NPT = NPAD // NS


_SC_MESH = plsc.VectorSubcoreMesh(
    core_axis_name="c", subcore_axis_name="s", num_cores=NC, num_subcores=NS
)


XCH = 632
GG = 64
GN = E // GG


def _gather_body(x_hbm, ei0_hbm, ei1_hbm, src_hbm, tgt_hbm,
                 x_sh, idx0a, idx1a, idx0b, idx1b,
                 buf0a, buf1a, buf0b, buf1b,
                 sa0, sa1, sb0, sb1):
    s = lax.axis_index("s")
    wid = lax.axis_index("c") * NS + s

    @pl.when(s < NS - 1)
    def _():
        pltpu.sync_copy(x_hbm.at[pl.ds(s * XCH, XCH)],
                        x_sh.at[pl.ds(s * XCH, XCH)])

    @pl.when(s == NS - 1)
    def _():
        pltpu.sync_copy(x_hbm.at[pl.ds((NS - 1) * XCH, N - (NS - 1) * XCH)],
                        x_sh.at[pl.ds((NS - 1) * XCH, N - (NS - 1) * XCH)])

    plsc.subcore_barrier()

    npairs = (GN + 2 * NW - 1) // (2 * NW)

    def eoff(j):
        return (wid + NW * j) * GG

    def valid(j):
        return wid + NW * j < GN

    def load_idx(j, i0, i1, s0, s1):
        @pl.when(valid(j))
        def _():
            pltpu.async_copy(ei0_hbm.at[pl.ds(eoff(j), GG)], i0, s0)
            pltpu.async_copy(ei1_hbm.at[pl.ds(eoff(j), GG)], i1, s1)

    def gather(j, i0, i1, b0, b1, s0, s1):
        @pl.when(valid(j))
        def _():
            pltpu.make_async_copy(ei0_hbm.at[pl.ds(eoff(j), GG)], i0,
                                  s0).wait()
            pltpu.make_async_copy(ei1_hbm.at[pl.ds(eoff(j), GG)], i1,
                                  s1).wait()
            pltpu.async_copy(x_sh.at[i0], b0, s0)
            pltpu.async_copy(x_sh.at[i1], b1, s1)

    def write(j, i0, i1, b0, b1, s0, s1):
        @pl.when(valid(j))
        def _():
            pltpu.make_async_copy(x_sh.at[i0], b0, s0).wait()
            pltpu.make_async_copy(x_sh.at[i1], b1, s1).wait()
            pltpu.async_copy(b0, src_hbm.at[pl.ds(eoff(j), GG)], s0)
            pltpu.async_copy(b1, tgt_hbm.at[pl.ds(eoff(j), GG)], s1)

    def drain(j, b0, b1, s0, s1):
        @pl.when(valid(j))
        def _():
            pltpu.make_async_copy(b0, src_hbm.at[pl.ds(eoff(j), GG)],
                                  s0).wait()
            pltpu.make_async_copy(b1, tgt_hbm.at[pl.ds(eoff(j), GG)],
                                  s1).wait()

    @pl.loop(0, npairs)
    def _(p):
        ja = 2 * p
        jb = 2 * p + 1
        load_idx(ja, idx0a, idx1a, sa0, sa1)
        load_idx(jb, idx0b, idx1b, sb0, sb1)
        gather(ja, idx0a, idx1a, buf0a, buf1a, sa0, sa1)
        gather(jb, idx0b, idx1b, buf0b, buf1b, sb0, sb1)
        write(ja, idx0a, idx1a, buf0a, buf1a, sa0, sa1)
        write(jb, idx0b, idx1b, buf0b, buf1b, sb0, sb1)
        drain(ja, buf0a, buf1a, sa0, sa1)
        drain(jb, buf0b, buf1b, sb0, sb1)


_sc_gather = functools.partial(
    pl.kernel,
    out_type=(
        jax.ShapeDtypeStruct((E, D), jnp.float32),
        jax.ShapeDtypeStruct((E, D), jnp.float32),
    ),
    mesh=_SC_MESH,
    scratch_types=[
        pltpu.VMEM_SHARED((N, D), jnp.float32),
        pltpu.VMEM((GG,), jnp.int32),
        pltpu.VMEM((GG,), jnp.int32),
        pltpu.VMEM((GG,), jnp.int32),
        pltpu.VMEM((GG,), jnp.int32),
        pltpu.VMEM((GG, D), jnp.float32),
        pltpu.VMEM((GG, D), jnp.float32),
        pltpu.VMEM((GG, D), jnp.float32),
        pltpu.VMEM((GG, D), jnp.float32),
        pltpu.SemaphoreType.DMA,
        pltpu.SemaphoreType.DMA,
        pltpu.SemaphoreType.DMA,
        pltpu.SemaphoreType.DMA,
    ],
)(_gather_body)


def _mlp_body(src_ref, tgt_ref, ea_ref, w1a_ref, w1b_ref, w1c_ref, b1_ref,
              w2_ref, b2_ref, mf_ref, mr_ref):
    diff = tgt_ref[...] - src_ref[...]
    d2 = diff * diff
    t1 = jnp.dot(diff, w1a_ref[...], preferred_element_type=jnp.float32)
    t2 = jnp.dot(d2, w1b_ref[...], preferred_element_type=jnp.float32)
    t3 = jnp.dot(ea_ref[...], w1c_ref[...], preferred_element_type=jnp.float32)
    even = t2 + b1_ref[...]
    odd = t1 + t3
    hf = jnp.maximum(even + odd, 0.0)
    hr = jnp.maximum(even - odd, 0.0)
    b2 = b2_ref[...]
    w2 = w2_ref[...]
    mf = jnp.maximum(jnp.dot(hf, w2, preferred_element_type=jnp.float32) + b2, 0.0)
    mr = jnp.maximum(jnp.dot(hr, w2, preferred_element_type=jnp.float32) + b2, 0.0)
    be = mf.shape[0]
    lane = jax.lax.broadcasted_iota(jnp.int32, (be, MW - M), 1)
    pad = jnp.where(lane == 0, 1.0, 0.0).astype(jnp.float32)
    mf_ref[...] = jnp.concatenate([mf, pad], axis=1)
    mr_ref[...] = jnp.concatenate([mr, pad], axis=1)


def _tc_mlp(src, tgt, ea, w1a, w1b, w1c, b1, w2, b2):
    BE = 1280
    nblk = E // BE
    return pl.pallas_call(
        _mlp_body,
        grid=(nblk,),
        in_specs=[
            pl.BlockSpec((BE, D), lambda i: (i, 0)),
            pl.BlockSpec((BE, D), lambda i: (i, 0)),
            pl.BlockSpec((BE, DE), lambda i: (i, 0)),
            pl.BlockSpec((D, H), lambda i: (0, 0)),
            pl.BlockSpec((D, H), lambda i: (0, 0)),
            pl.BlockSpec((DE, H), lambda i: (0, 0)),
            pl.BlockSpec((1, H), lambda i: (0, 0)),
            pl.BlockSpec((H, M), lambda i: (0, 0)),
            pl.BlockSpec((1, M), lambda i: (0, 0)),
        ],
        out_specs=[
            pl.BlockSpec((BE, MW), lambda i: (i, 0)),
            pl.BlockSpec((BE, MW), lambda i: (i, 0)),
        ],
        out_shape=[
            jax.ShapeDtypeStruct((E, MW), jnp.float32),
            jax.ShapeDtypeStruct((E, MW), jnp.float32),
        ],
    )(src, tgt, ea, w1a, w1b, w1c, b1, w2, b2)


def _scatter_body(mf_hbm, mr_hbm, ei0_hbm, ei1_hbm, z_hbm,
                  s_hbm, s_sh, idx0, idx1, vf, vr):
    c = lax.axis_index("c")
    s = lax.axis_index("s")

    pltpu.sync_copy(z_hbm.at[pl.ds(s * NPT, NPT)], s_sh.at[pl.ds(s * NPT, NPT)])
    plsc.subcore_barrier()

    gpc = G // NC
    ngroups = (gpc + NS - 1) // NS

    @pl.loop(0, ngroups)
    def _(j):
        gl = s + NS * j

        @pl.when(gl < gpc)
        def _():
            g = c * gpc + gl
            e0 = g * GSZ
            pltpu.sync_copy(ei0_hbm.at[pl.ds(e0, GSZ)], idx0)
            pltpu.sync_copy(ei1_hbm.at[pl.ds(e0, GSZ)], idx1)
            pltpu.sync_copy(mf_hbm.at[pl.ds(e0, GSZ)], vf)
            pltpu.sync_copy(mr_hbm.at[pl.ds(e0, GSZ)], vr)
            pltpu.sync_copy(vf, s_sh.at[idx1], add=True)
            pltpu.sync_copy(vr, s_sh.at[idx0], add=True)

    plsc.subcore_barrier()
    pltpu.sync_copy(s_sh.at[pl.ds(s * NPT, NPT)], s_hbm.at[c, pl.ds(s * NPT, NPT)])


_sc_scatter = functools.partial(
    pl.kernel,
    out_type=jax.ShapeDtypeStruct((NC, NPAD, MW), jnp.float32),
    mesh=_SC_MESH,
    scratch_types=[
        pltpu.VMEM_SHARED((NPAD, MW), jnp.float32),
        pltpu.VMEM((GSZ,), jnp.int32),
        pltpu.VMEM((GSZ,), jnp.int32),
        pltpu.VMEM((GSZ, MW), jnp.float32),
        pltpu.VMEM((GSZ, MW), jnp.float32),
    ],
)(_scatter_body)


def _final_body(x_ref, sp_ref, wua_ref, wub_ref, wuc_ref, bu_ref, out_ref):
    x = x_ref[...]
    acc = sp_ref[0] + sp_ref[1]
    ssum = acc[:, :M]
    cnt = jnp.maximum(acc[:, M:M + 1], 1.0)
    aggr = ssum / cnt
    out = jnp.dot(x, wua_ref[...], preferred_element_type=jnp.float32)
    out += jnp.dot(x * x, wub_ref[...], preferred_element_type=jnp.float32)
    out += jnp.dot(aggr, wuc_ref[...], preferred_element_type=jnp.float32)
    out_ref[...] = out + bu_ref[...]


def _tc_final(x, sp, wua, wub, wuc, bu):
    return pl.pallas_call(
        _final_body,
        out_shape=jax.ShapeDtypeStruct((N, DOUT), jnp.float32),
    )(x, sp, wua, wub, wuc, bu)


def kernel(x, edge_index, edge_attr, W1, b1, W2, b2, Wu, bu):
    ei0 = edge_index[0]
    ei1 = edge_index[1]

    w1a = W1[:D]
    w1b = W1[D:2 * D]
    w1c = W1[2 * D:]
    b1r = b1.reshape(1, H)
    b2r = b2.reshape(1, M)

    src, tgt = _sc_gather(x, ei0, ei1)
    mf, mr = _tc_mlp(src, tgt, edge_attr, w1a, w1b, w1c, b1r, W2, b2r)

    z = jnp.zeros((NPAD, MW), jnp.float32)
    sp = _sc_scatter(mf, mr, ei0, ei1, z)
    sp = sp[:, :N]

    wua = Wu[:D]
    wub = Wu[D:2 * D]
    wuc = Wu[2 * D:]
    bur = bu.reshape(1, DOUT)
    return _tc_final(x, sp, wua, wub, wuc, bur)


# speedup vs baseline: 8.6863x; 1.1360x over previous
import functools

import jax
import jax.numpy as jnp
from jax import lax
from jax.experimental import pallas as pl
from jax.experimental.pallas import tpu as pltpu
from jax.experimental.pallas import tpu_sc as plsc

N = 10000
E = 320000
D = 128
DE = 16
H = 128
M = 64
MW = 128
DOUT = 128

NC = 2
NS = 16
NW = NC * NS
GSZ = 128
G = E // GSZ
NPAD = 10240
NPT = NPAD // NS


_SC_MESH = plsc.VectorSubcoreMesh(
    core_axis_name="c", subcore_axis_name="s", num_cores=NC, num_subcores=NS
)


XCH = 632
GG = 64
GN = E // GG


def _gather_body(x_hbm, ei0_hbm, ei1_hbm, src_hbm, tgt_hbm,
                 x_sh, idx0a, idx1a, idx0b, idx1b,
                 buf0a, buf1a, buf0b, buf1b,
                 sa0, sa1, sb0, sb1):
    s = lax.axis_index("s")
    wid = lax.axis_index("c") * NS + s

    @pl.when(s < NS - 1)
    def _():
        pltpu.sync_copy(x_hbm.at[pl.ds(s * XCH, XCH)],
                        x_sh.at[pl.ds(s * XCH, XCH)])

    @pl.when(s == NS - 1)
    def _():
        pltpu.sync_copy(x_hbm.at[pl.ds((NS - 1) * XCH, N - (NS - 1) * XCH)],
                        x_sh.at[pl.ds((NS - 1) * XCH, N - (NS - 1) * XCH)])

    plsc.subcore_barrier()

    npairs = (GN + 2 * NW - 1) // (2 * NW)

    def eoff(j):
        return (wid + NW * j) * GG

    def valid(j):
        return wid + NW * j < GN

    def load_idx(j, i0, i1, s0, s1):
        @pl.when(valid(j))
        def _():
            pltpu.async_copy(ei0_hbm.at[pl.ds(eoff(j), GG)], i0, s0)
            pltpu.async_copy(ei1_hbm.at[pl.ds(eoff(j), GG)], i1, s1)

    def gather(j, i0, i1, b0, b1, s0, s1):
        @pl.when(valid(j))
        def _():
            pltpu.make_async_copy(ei0_hbm.at[pl.ds(eoff(j), GG)], i0,
                                  s0).wait()
            pltpu.make_async_copy(ei1_hbm.at[pl.ds(eoff(j), GG)], i1,
                                  s1).wait()
            pltpu.async_copy(x_sh.at[i0], b0, s0)
            pltpu.async_copy(x_sh.at[i1], b1, s1)

    def write(j, i0, i1, b0, b1, s0, s1):
        @pl.when(valid(j))
        def _():
            pltpu.make_async_copy(x_sh.at[i0], b0, s0).wait()
            pltpu.make_async_copy(x_sh.at[i1], b1, s1).wait()
            pltpu.async_copy(b0, src_hbm.at[pl.ds(eoff(j), GG)], s0)
            pltpu.async_copy(b1, tgt_hbm.at[pl.ds(eoff(j), GG)], s1)

    def drain(j, b0, b1, s0, s1):
        @pl.when(valid(j))
        def _():
            pltpu.make_async_copy(b0, src_hbm.at[pl.ds(eoff(j), GG)],
                                  s0).wait()
            pltpu.make_async_copy(b1, tgt_hbm.at[pl.ds(eoff(j), GG)],
                                  s1).wait()

    @pl.loop(0, npairs)
    def _(p):
        ja = 2 * p
        jb = 2 * p + 1
        load_idx(ja, idx0a, idx1a, sa0, sa1)
        load_idx(jb, idx0b, idx1b, sb0, sb1)
        gather(ja, idx0a, idx1a, buf0a, buf1a, sa0, sa1)
        gather(jb, idx0b, idx1b, buf0b, buf1b, sb0, sb1)
        write(ja, idx0a, idx1a, buf0a, buf1a, sa0, sa1)
        write(jb, idx0b, idx1b, buf0b, buf1b, sb0, sb1)
        drain(ja, buf0a, buf1a, sa0, sa1)
        drain(jb, buf0b, buf1b, sb0, sb1)


_sc_gather = functools.partial(
    pl.kernel,
    out_type=(
        jax.ShapeDtypeStruct((E, D), jnp.float32),
        jax.ShapeDtypeStruct((E, D), jnp.float32),
    ),
    mesh=_SC_MESH,
    scratch_types=[
        pltpu.VMEM_SHARED((N, D), jnp.float32),
        pltpu.VMEM((GG,), jnp.int32),
        pltpu.VMEM((GG,), jnp.int32),
        pltpu.VMEM((GG,), jnp.int32),
        pltpu.VMEM((GG,), jnp.int32),
        pltpu.VMEM((GG, D), jnp.float32),
        pltpu.VMEM((GG, D), jnp.float32),
        pltpu.VMEM((GG, D), jnp.float32),
        pltpu.VMEM((GG, D), jnp.float32),
        pltpu.SemaphoreType.DMA,
        pltpu.SemaphoreType.DMA,
        pltpu.SemaphoreType.DMA,
        pltpu.SemaphoreType.DMA,
    ],
)(_gather_body)


def _mlp_body(src_ref, tgt_ref, ea_ref, w1a_ref, w1b_ref, w1c_ref, b1_ref,
              w2_ref, b2_ref, mf_ref, mr_ref):
    diff = tgt_ref[...] - src_ref[...]
    d2 = diff * diff
    t1 = jnp.dot(diff, w1a_ref[...], preferred_element_type=jnp.float32)
    t2 = jnp.dot(d2, w1b_ref[...], preferred_element_type=jnp.float32)
    t3 = jnp.dot(ea_ref[...], w1c_ref[...], preferred_element_type=jnp.float32)
    even = t2 + b1_ref[...]
    odd = t1 + t3
    hf = jnp.maximum(even + odd, 0.0)
    hr = jnp.maximum(even - odd, 0.0)
    b2 = b2_ref[...]
    w2 = w2_ref[...]
    mf = jnp.maximum(jnp.dot(hf, w2, preferred_element_type=jnp.float32) + b2, 0.0)
    mr = jnp.maximum(jnp.dot(hr, w2, preferred_element_type=jnp.float32) + b2, 0.0)
    be = mf.shape[0]
    lane = jax.lax.broadcasted_iota(jnp.int32, (be, MW - M), 1)
    pad = jnp.where(lane == 0, 1.0, 0.0).astype(jnp.float32)
    mf_ref[...] = jnp.concatenate([mf, pad], axis=1)
    mr_ref[...] = jnp.concatenate([mr, pad], axis=1)


def _tc_mlp(src, tgt, ea, w1a, w1b, w1c, b1, w2, b2):
    BE = 1280
    nblk = E // BE
    return pl.pallas_call(
        _mlp_body,
        grid=(nblk,),
        in_specs=[
            pl.BlockSpec((BE, D), lambda i: (i, 0)),
            pl.BlockSpec((BE, D), lambda i: (i, 0)),
            pl.BlockSpec((BE, DE), lambda i: (i, 0)),
            pl.BlockSpec((D, H), lambda i: (0, 0)),
            pl.BlockSpec((D, H), lambda i: (0, 0)),
            pl.BlockSpec((DE, H), lambda i: (0, 0)),
            pl.BlockSpec((1, H), lambda i: (0, 0)),
            pl.BlockSpec((H, M), lambda i: (0, 0)),
            pl.BlockSpec((1, M), lambda i: (0, 0)),
        ],
        out_specs=[
            pl.BlockSpec((BE, MW), lambda i: (i, 0)),
            pl.BlockSpec((BE, MW), lambda i: (i, 0)),
        ],
        out_shape=[
            jax.ShapeDtypeStruct((E, MW), jnp.float32),
            jax.ShapeDtypeStruct((E, MW), jnp.float32),
        ],
    )(src, tgt, ea, w1a, w1b, w1c, b1, w2, b2)


SG = 64
SGN = E // SG
SPC = SGN // NC


def _scatter_body(mf_hbm, mr_hbm, ei0_hbm, ei1_hbm, z_hbm, s_hbm,
                  s_sh, i0a, i1a, i0b, i1b, vfa, vra, vfb, vrb,
                  qa, qb, ra, rb):
    c = lax.axis_index("c")
    s = lax.axis_index("s")

    pltpu.sync_copy(z_hbm.at[pl.ds(s * NPT, NPT)], s_sh.at[pl.ds(s * NPT, NPT)])
    plsc.subcore_barrier()

    npairs = (SPC + 2 * NS - 1) // (2 * NS)

    def eoff(j):
        return (c * SPC + s + NS * j) * SG

    def valid(j):
        return s + NS * j < SPC

    def load(j, i0, i1, vf, vr, q):
        @pl.when(valid(j))
        def _():
            e0 = eoff(j)
            pltpu.async_copy(ei0_hbm.at[pl.ds(e0, SG)], i0, q)
            pltpu.async_copy(ei1_hbm.at[pl.ds(e0, SG)], i1, q)
            pltpu.async_copy(mf_hbm.at[pl.ds(e0, SG)], vf, q)
            pltpu.async_copy(mr_hbm.at[pl.ds(e0, SG)], vr, q)

    def add(j, i0, i1, vf, vr, q, r):
        @pl.when(valid(j))
        def _():
            e0 = eoff(j)
            pltpu.make_async_copy(ei0_hbm.at[pl.ds(e0, SG)], i0, q).wait()
            pltpu.make_async_copy(ei1_hbm.at[pl.ds(e0, SG)], i1, q).wait()
            pltpu.make_async_copy(mf_hbm.at[pl.ds(e0, SG)], vf, q).wait()
            pltpu.make_async_copy(mr_hbm.at[pl.ds(e0, SG)], vr, q).wait()
            pltpu.async_copy(vf, s_sh.at[i1], r, add=True)
            pltpu.async_copy(vr, s_sh.at[i0], r, add=True)

    def drain(j, i0, i1, vf, vr, r):
        @pl.when(valid(j))
        def _():
            pltpu.make_async_copy(vf, s_sh.at[i1], r).wait()
            pltpu.make_async_copy(vr, s_sh.at[i0], r).wait()

    @pl.loop(0, npairs)
    def _(p):
        ja = 2 * p
        jb = 2 * p + 1
        load(ja, i0a, i1a, vfa, vra, qa)
        load(jb, i0b, i1b, vfb, vrb, qb)
        add(ja, i0a, i1a, vfa, vra, qa, ra)
        add(jb, i0b, i1b, vfb, vrb, qb, rb)
        drain(ja, i0a, i1a, vfa, vra, ra)
        drain(jb, i0b, i1b, vfb, vrb, rb)

    plsc.subcore_barrier()
    pltpu.sync_copy(s_sh.at[pl.ds(s * NPT, NPT)], s_hbm.at[c, pl.ds(s * NPT, NPT)])


_sc_scatter = functools.partial(
    pl.kernel,
    out_type=jax.ShapeDtypeStruct((NC, NPAD, MW), jnp.float32),
    mesh=_SC_MESH,
    scratch_types=[
        pltpu.VMEM_SHARED((NPAD, MW), jnp.float32),
        pltpu.VMEM((SG,), jnp.int32),
        pltpu.VMEM((SG,), jnp.int32),
        pltpu.VMEM((SG,), jnp.int32),
        pltpu.VMEM((SG,), jnp.int32),
        pltpu.VMEM((SG, MW), jnp.float32),
        pltpu.VMEM((SG, MW), jnp.float32),
        pltpu.VMEM((SG, MW), jnp.float32),
        pltpu.VMEM((SG, MW), jnp.float32),
        pltpu.SemaphoreType.DMA,
        pltpu.SemaphoreType.DMA,
        pltpu.SemaphoreType.DMA,
        pltpu.SemaphoreType.DMA,
    ],
)(_scatter_body)


def _final_body(x_ref, sp_ref, wua_ref, wub_ref, wuc_ref, bu_ref, out_ref):
    x = x_ref[...]
    acc = sp_ref[0] + sp_ref[1]
    ssum = acc[:, :M]
    cnt = jnp.maximum(acc[:, M:M + 1], 1.0)
    aggr = ssum / cnt
    out = jnp.dot(x, wua_ref[...], preferred_element_type=jnp.float32)
    out += jnp.dot(x * x, wub_ref[...], preferred_element_type=jnp.float32)
    out += jnp.dot(aggr, wuc_ref[...], preferred_element_type=jnp.float32)
    out_ref[...] = out + bu_ref[...]


def _tc_final(x, sp, wua, wub, wuc, bu):
    return pl.pallas_call(
        _final_body,
        out_shape=jax.ShapeDtypeStruct((N, DOUT), jnp.float32),
    )(x, sp, wua, wub, wuc, bu)


def kernel(x, edge_index, edge_attr, W1, b1, W2, b2, Wu, bu):
    ei0 = edge_index[0]
    ei1 = edge_index[1]

    w1a = W1[:D]
    w1b = W1[D:2 * D]
    w1c = W1[2 * D:]
    b1r = b1.reshape(1, H)
    b2r = b2.reshape(1, M)

    src, tgt = _sc_gather(x, ei0, ei1)
    mf, mr = _tc_mlp(src, tgt, edge_attr, w1a, w1b, w1c, b1r, W2, b2r)

    z = jnp.zeros((NPAD, MW), jnp.float32)
    sp = _sc_scatter(mf, mr, ei0, ei1, z)
    sp = sp[:, :N]

    wua = Wu[:D]
    wub = Wu[D:2 * D]
    wuc = Wu[2 * D:]
    bur = bu.reshape(1, DOUT)
    return _tc_final(x, sp, wua, wub, wuc, bur)


# speedup vs baseline: 8.8090x; 1.0141x over previous
import functools

import jax
import jax.numpy as jnp
from jax import lax
from jax.experimental import pallas as pl
from jax.experimental.pallas import tpu as pltpu
from jax.experimental.pallas import tpu_sc as plsc

N = 10000
E = 320000
D = 128
DE = 16
H = 128
M = 64
MW = 128
DOUT = 128

NC = 2
NS = 16
NW = NC * NS
GSZ = 128
G = E // GSZ
NPAD = 10112
NPT = NPAD // NS


_SC_MESH = plsc.VectorSubcoreMesh(
    core_axis_name="c", subcore_axis_name="s", num_cores=NC, num_subcores=NS
)


XCH = 632
GG = 64
GN = E // GG


def _gather_body(x_hbm, ei0_hbm, ei1_hbm, src_hbm, tgt_hbm,
                 x_sh, idx0a, idx1a, idx0b, idx1b,
                 buf0a, buf1a, buf0b, buf1b,
                 sa0, sa1, sb0, sb1):
    s = lax.axis_index("s")
    wid = lax.axis_index("c") * NS + s

    @pl.when(s < NS - 1)
    def _():
        pltpu.sync_copy(x_hbm.at[pl.ds(s * XCH, XCH)],
                        x_sh.at[pl.ds(s * XCH, XCH)])

    @pl.when(s == NS - 1)
    def _():
        pltpu.sync_copy(x_hbm.at[pl.ds((NS - 1) * XCH, N - (NS - 1) * XCH)],
                        x_sh.at[pl.ds((NS - 1) * XCH, N - (NS - 1) * XCH)])

    plsc.subcore_barrier()

    npairs = (GN + 2 * NW - 1) // (2 * NW)

    def eoff(j):
        return (wid + NW * j) * GG

    def valid(j):
        return wid + NW * j < GN

    def load_idx(j, i0, i1, s0, s1):
        @pl.when(valid(j))
        def _():
            pltpu.async_copy(ei0_hbm.at[pl.ds(eoff(j), GG)], i0, s0)
            pltpu.async_copy(ei1_hbm.at[pl.ds(eoff(j), GG)], i1, s1)

    def gather(j, i0, i1, b0, b1, s0, s1):
        @pl.when(valid(j))
        def _():
            pltpu.make_async_copy(ei0_hbm.at[pl.ds(eoff(j), GG)], i0,
                                  s0).wait()
            pltpu.make_async_copy(ei1_hbm.at[pl.ds(eoff(j), GG)], i1,
                                  s1).wait()
            pltpu.async_copy(x_sh.at[i0], b0, s0)
            pltpu.async_copy(x_sh.at[i1], b1, s1)

    def write(j, i0, i1, b0, b1, s0, s1):
        @pl.when(valid(j))
        def _():
            pltpu.make_async_copy(x_sh.at[i0], b0, s0).wait()
            pltpu.make_async_copy(x_sh.at[i1], b1, s1).wait()
            pltpu.async_copy(b0, src_hbm.at[pl.ds(eoff(j), GG)], s0)
            pltpu.async_copy(b1, tgt_hbm.at[pl.ds(eoff(j), GG)], s1)

    def drain(j, b0, b1, s0, s1):
        @pl.when(valid(j))
        def _():
            pltpu.make_async_copy(b0, src_hbm.at[pl.ds(eoff(j), GG)],
                                  s0).wait()
            pltpu.make_async_copy(b1, tgt_hbm.at[pl.ds(eoff(j), GG)],
                                  s1).wait()

    @pl.loop(0, npairs)
    def _(p):
        ja = 2 * p
        jb = 2 * p + 1
        load_idx(ja, idx0a, idx1a, sa0, sa1)
        load_idx(jb, idx0b, idx1b, sb0, sb1)
        gather(ja, idx0a, idx1a, buf0a, buf1a, sa0, sa1)
        gather(jb, idx0b, idx1b, buf0b, buf1b, sb0, sb1)
        write(ja, idx0a, idx1a, buf0a, buf1a, sa0, sa1)
        write(jb, idx0b, idx1b, buf0b, buf1b, sb0, sb1)
        drain(ja, buf0a, buf1a, sa0, sa1)
        drain(jb, buf0b, buf1b, sb0, sb1)


_sc_gather = functools.partial(
    pl.kernel,
    out_type=(
        jax.ShapeDtypeStruct((E, D), jnp.float32),
        jax.ShapeDtypeStruct((E, D), jnp.float32),
    ),
    mesh=_SC_MESH,
    scratch_types=[
        pltpu.VMEM_SHARED((N, D), jnp.float32),
        pltpu.VMEM((GG,), jnp.int32),
        pltpu.VMEM((GG,), jnp.int32),
        pltpu.VMEM((GG,), jnp.int32),
        pltpu.VMEM((GG,), jnp.int32),
        pltpu.VMEM((GG, D), jnp.float32),
        pltpu.VMEM((GG, D), jnp.float32),
        pltpu.VMEM((GG, D), jnp.float32),
        pltpu.VMEM((GG, D), jnp.float32),
        pltpu.SemaphoreType.DMA,
        pltpu.SemaphoreType.DMA,
        pltpu.SemaphoreType.DMA,
        pltpu.SemaphoreType.DMA,
    ],
)(_gather_body)


def _mlp_body(src_ref, tgt_ref, ea_ref, w1a_ref, w1b_ref, w1c_ref, b1_ref,
              w2_ref, b2_ref, mf_ref, mr_ref):
    diff = tgt_ref[...] - src_ref[...]
    d2 = diff * diff
    t1 = jnp.dot(diff, w1a_ref[...], preferred_element_type=jnp.float32)
    t2 = jnp.dot(d2, w1b_ref[...], preferred_element_type=jnp.float32)
    t3 = jnp.dot(ea_ref[...], w1c_ref[...], preferred_element_type=jnp.float32)
    even = t2 + b1_ref[...]
    odd = t1 + t3
    hf = jnp.maximum(even + odd, 0.0)
    hr = jnp.maximum(even - odd, 0.0)
    b2 = b2_ref[...]
    w2 = w2_ref[...]
    mf = jnp.maximum(jnp.dot(hf, w2, preferred_element_type=jnp.float32) + b2, 0.0)
    mr = jnp.maximum(jnp.dot(hr, w2, preferred_element_type=jnp.float32) + b2, 0.0)
    be = mf.shape[0]
    lane = jax.lax.broadcasted_iota(jnp.int32, (be, MW - M), 1)
    pad = jnp.where(lane == 0, 1.0, 0.0).astype(jnp.float32)
    mf_ref[...] = jnp.concatenate([mf, pad], axis=1)
    mr_ref[...] = jnp.concatenate([mr, pad], axis=1)


def _tc_mlp(src, tgt, ea, w1a, w1b, w1c, b1, w2, b2):
    BE = 1280
    nblk = E // BE
    return pl.pallas_call(
        _mlp_body,
        grid=(nblk,),
        in_specs=[
            pl.BlockSpec((BE, D), lambda i: (i, 0)),
            pl.BlockSpec((BE, D), lambda i: (i, 0)),
            pl.BlockSpec((BE, DE), lambda i: (i, 0)),
            pl.BlockSpec((D, H), lambda i: (0, 0)),
            pl.BlockSpec((D, H), lambda i: (0, 0)),
            pl.BlockSpec((DE, H), lambda i: (0, 0)),
            pl.BlockSpec((1, H), lambda i: (0, 0)),
            pl.BlockSpec((H, M), lambda i: (0, 0)),
            pl.BlockSpec((1, M), lambda i: (0, 0)),
        ],
        out_specs=[
            pl.BlockSpec((BE, MW), lambda i: (i, 0)),
            pl.BlockSpec((BE, MW), lambda i: (i, 0)),
        ],
        out_shape=[
            jax.ShapeDtypeStruct((E, MW), jnp.float32),
            jax.ShapeDtypeStruct((E, MW), jnp.float32),
        ],
    )(src, tgt, ea, w1a, w1b, w1c, b1, w2, b2)


SG = 64
SGN = E // SG
SPC = SGN // NC


def _scatter_body(mf_hbm, mr_hbm, ei0_hbm, ei1_hbm, z_hbm, s_hbm,
                  s_sh, i0a, i1a, i0b, i1b, i0c, i1c,
                  vfa, vra, vfb, vrb, vfc, vrc,
                  qa, qb, qc, ra, rb, rc):
    c = lax.axis_index("c")
    s = lax.axis_index("s")

    pltpu.sync_copy(z_hbm.at[pl.ds(s * NPT, NPT)], s_sh.at[pl.ds(s * NPT, NPT)])
    plsc.subcore_barrier()

    ntrips = (SPC + 3 * NS - 1) // (3 * NS)

    def eoff(j):
        return (c * SPC + s + NS * j) * SG

    def valid(j):
        return s + NS * j < SPC

    def load(j, i0, i1, vf, vr, q):
        @pl.when(valid(j))
        def _():
            e0 = eoff(j)
            pltpu.async_copy(ei0_hbm.at[pl.ds(e0, SG)], i0, q)
            pltpu.async_copy(ei1_hbm.at[pl.ds(e0, SG)], i1, q)
            pltpu.async_copy(mf_hbm.at[pl.ds(e0, SG)], vf, q)
            pltpu.async_copy(mr_hbm.at[pl.ds(e0, SG)], vr, q)

    def add(j, i0, i1, vf, vr, q, r):
        @pl.when(valid(j))
        def _():
            e0 = eoff(j)
            pltpu.make_async_copy(ei0_hbm.at[pl.ds(e0, SG)], i0, q).wait()
            pltpu.make_async_copy(ei1_hbm.at[pl.ds(e0, SG)], i1, q).wait()
            pltpu.make_async_copy(mf_hbm.at[pl.ds(e0, SG)], vf, q).wait()
            pltpu.make_async_copy(mr_hbm.at[pl.ds(e0, SG)], vr, q).wait()
            pltpu.async_copy(vf, s_sh.at[i1], r, add=True)
            pltpu.async_copy(vr, s_sh.at[i0], r, add=True)

    def drain(j, i0, i1, vf, vr, r):
        @pl.when(valid(j))
        def _():
            pltpu.make_async_copy(vf, s_sh.at[i1], r).wait()
            pltpu.make_async_copy(vr, s_sh.at[i0], r).wait()

    @pl.loop(0, ntrips)
    def _(p):
        ja = 3 * p
        jb = 3 * p + 1
        jc = 3 * p + 2
        load(ja, i0a, i1a, vfa, vra, qa)
        load(jb, i0b, i1b, vfb, vrb, qb)
        load(jc, i0c, i1c, vfc, vrc, qc)
        add(ja, i0a, i1a, vfa, vra, qa, ra)
        add(jb, i0b, i1b, vfb, vrb, qb, rb)
        add(jc, i0c, i1c, vfc, vrc, qc, rc)
        drain(ja, i0a, i1a, vfa, vra, ra)
        drain(jb, i0b, i1b, vfb, vrb, rb)
        drain(jc, i0c, i1c, vfc, vrc, rc)

    plsc.subcore_barrier()
    pltpu.sync_copy(s_sh.at[pl.ds(s * NPT, NPT)], s_hbm.at[c, pl.ds(s * NPT, NPT)])


_sc_scatter = functools.partial(
    pl.kernel,
    out_type=jax.ShapeDtypeStruct((NC, NPAD, MW), jnp.float32),
    mesh=_SC_MESH,
    scratch_types=[
        pltpu.VMEM_SHARED((NPAD, MW), jnp.float32),
        pltpu.VMEM((SG,), jnp.int32),
        pltpu.VMEM((SG,), jnp.int32),
        pltpu.VMEM((SG,), jnp.int32),
        pltpu.VMEM((SG,), jnp.int32),
        pltpu.VMEM((SG,), jnp.int32),
        pltpu.VMEM((SG,), jnp.int32),
        pltpu.VMEM((SG, MW), jnp.float32),
        pltpu.VMEM((SG, MW), jnp.float32),
        pltpu.VMEM((SG, MW), jnp.float32),
        pltpu.VMEM((SG, MW), jnp.float32),
        pltpu.VMEM((SG, MW), jnp.float32),
        pltpu.VMEM((SG, MW), jnp.float32),
        pltpu.SemaphoreType.DMA,
        pltpu.SemaphoreType.DMA,
        pltpu.SemaphoreType.DMA,
        pltpu.SemaphoreType.DMA,
        pltpu.SemaphoreType.DMA,
        pltpu.SemaphoreType.DMA,
    ],
)(_scatter_body)


def _final_body(x_ref, sp_ref, wua_ref, wub_ref, wuc_ref, bu_ref, out_ref):
    x = x_ref[...]
    acc = sp_ref[0] + sp_ref[1]
    ssum = acc[:, :M]
    cnt = jnp.maximum(acc[:, M:M + 1], 1.0)
    aggr = ssum / cnt
    out = jnp.dot(x, wua_ref[...], preferred_element_type=jnp.float32)
    out += jnp.dot(x * x, wub_ref[...], preferred_element_type=jnp.float32)
    out += jnp.dot(aggr, wuc_ref[...], preferred_element_type=jnp.float32)
    out_ref[...] = out + bu_ref[...]


def _tc_final(x, sp, wua, wub, wuc, bu):
    return pl.pallas_call(
        _final_body,
        out_shape=jax.ShapeDtypeStruct((N, DOUT), jnp.float32),
    )(x, sp, wua, wub, wuc, bu)


def kernel(x, edge_index, edge_attr, W1, b1, W2, b2, Wu, bu):
    ei0 = edge_index[0]
    ei1 = edge_index[1]

    w1a = W1[:D]
    w1b = W1[D:2 * D]
    w1c = W1[2 * D:]
    b1r = b1.reshape(1, H)
    b2r = b2.reshape(1, M)

    src, tgt = _sc_gather(x, ei0, ei1)
    mf, mr = _tc_mlp(src, tgt, edge_attr, w1a, w1b, w1c, b1r, W2, b2r)

    z = jnp.zeros((NPAD, MW), jnp.float32)
    sp = _sc_scatter(mf, mr, ei0, ei1, z)
    sp = sp[:, :N]

    wua = Wu[:D]
    wub = Wu[D:2 * D]
    wuc = Wu[2 * D:]
    bur = bu.reshape(1, DOUT)
    return _tc_final(x, sp, wua, wub, wuc, bur)
